# Initial kernel scaffold; baseline (speedup 1.0000x reference)
import jax
import jax.numpy as jnp
from jax.experimental import pallas as pl


def kernel(x, edge_index, W1, b1, W2, b2):
    raise NotImplementedError("write your pallas kernel here")


# speedup vs baseline: 23.1780x; 23.1780x over previous
import functools

import jax
import jax.numpy as jnp
from jax import lax
from jax.experimental import pallas as pl
from jax.experimental.pallas import tpu as pltpu
from jax.experimental.pallas import tpu_sc as plsc

N = 10000
NPAD = 10240
E = 320000
CHUNK = 125
NTILES = 16
RPT = NPAD // NTILES
HH = 128

AGG_CHUNKS = E // (NTILES * CHUNK)
DEG_CHUNKS = E // (2 * NTILES * CHUNK)

_sc_mesh = plsc.VectorSubcoreMesh(core_axis_name="c", subcore_axis_name="s")


def _deg_body(dst_hbm, zeros_hbm, ones_hbm, degp_hbm, idx_v, ones_v, deg_sh):
    c = lax.axis_index("c")
    s = lax.axis_index("s")
    pltpu.sync_copy(zeros_hbm, deg_sh.at[pl.ds(s * RPT, RPT)])
    pltpu.sync_copy(dst_hbm.at[c, s], idx_v)
    pltpu.sync_copy(ones_hbm, ones_v)
    plsc.subcore_barrier()

    def body(j, carry):
        pltpu.sync_copy(ones_v.at[pl.ds(0, CHUNK)], deg_sh.at[idx_v.at[j]],
                        add=True)
        return carry

    lax.fori_loop(0, DEG_CHUNKS, body, 0)
    plsc.subcore_barrier()
    pltpu.sync_copy(deg_sh.at[pl.ds(s * RPT, RPT)],
                    degp_hbm.at[c, pl.ds(s * RPT, RPT)])


_deg_kernel = functools.partial(
    pl.kernel,
    out_type=jax.ShapeDtypeStruct((2, NPAD), jnp.float32),
    mesh=_sc_mesh,
    scratch_types=[
        pltpu.VMEM((DEG_CHUNKS, CHUNK), jnp.int32),
        pltpu.VMEM((CHUNK,), jnp.float32),
        pltpu.VMEM_SHARED((NPAD,), jnp.float32),
    ],
)(_deg_body)


GRP = 16
NGRP = AGG_CHUNKS // GRP


def _agg_body(tbl_hbm, src_hbm, dst_hbm, out_hbm, src_v, dst_v, buf, sem,
              agg_sh):
    c = lax.axis_index("c")
    s = lax.axis_index("s")
    r0 = s * RPT
    pltpu.sync_copy(tbl_hbm.at[c, pl.ds(r0, RPT)], agg_sh.at[pl.ds(r0, RPT)])
    plsc.subcore_barrier()

    tbl_c = tbl_hbm.at[c]

    def issue(j, b):
        pltpu.async_copy(tbl_c.at[src_v.at[j]], buf.at[b], sem)

    def group(g, carry):
        pltpu.sync_copy(src_hbm.at[s, pl.ds(g * GRP, GRP)], src_v)
        pltpu.sync_copy(dst_hbm.at[s, pl.ds(g * GRP, GRP)], dst_v)
        issue(0, 0)
        issue(1, 1)

        def body(j2, carry2):
            for b in range(2):
                j = j2 * 2 + b
                pltpu.make_async_copy(tbl_c.at[src_v.at[j]], buf.at[b],
                                      sem).wait()

                @pl.when(j + 2 < GRP)
                def _():
                    issue(j + 2, b)

                pltpu.sync_copy(buf.at[b], agg_sh.at[dst_v.at[j]], add=True)
            return carry2

        lax.fori_loop(0, GRP // 2, body, 0)
        return carry

    lax.fori_loop(0, NGRP, group, 0)
    plsc.subcore_barrier()
    pltpu.sync_copy(agg_sh.at[pl.ds(r0, RPT)], out_hbm.at[c, pl.ds(r0, RPT)])


_agg_kernel = functools.partial(
    pl.kernel,
    out_type=jax.ShapeDtypeStruct((2, NPAD, HH), jnp.float32),
    mesh=_sc_mesh,
    scratch_types=[
        pltpu.VMEM((GRP, CHUNK), jnp.int32),
        pltpu.VMEM((GRP, CHUNK), jnp.int32),
        pltpu.VMEM((2, CHUNK, HH), jnp.float32),
        pltpu.SemaphoreType.DMA,
        pltpu.VMEM_SHARED((NPAD, HH), jnp.float32),
    ],
)(_agg_body)


_BM = 1000


def _mm1_body(dinv_ref, x_ref, w_ref, h_ref):
    h = jnp.dot(x_ref[...], w_ref[...], preferred_element_type=jnp.float32)
    h_ref[0] = h * dinv_ref[...]


def _mm1_call(dinv, x, W1):
    return pl.pallas_call(
        _mm1_body,
        grid=(N // _BM, 2),
        in_specs=[
            pl.BlockSpec((_BM, 1), lambda i, j: (i, 0)),
            pl.BlockSpec((_BM, 128), lambda i, j: (i, 0)),
            pl.BlockSpec((128, HH), lambda i, j: (0, j)),
        ],
        out_specs=pl.BlockSpec((1, _BM, HH), lambda i, j: (j, i, 0)),
        out_shape=jax.ShapeDtypeStruct((2, N, HH), jnp.float32),
    )(dinv, x, W1)


def _mm2_body(agg_ref, dinv_ref, b_ref, w_ref, out_ref):
    dinv = dinv_ref[...]
    a0 = jnp.maximum(agg_ref[0] * dinv + b_ref[0, :][None, :], 0.0)
    a1 = jnp.maximum(agg_ref[1] * dinv + b_ref[1, :][None, :], 0.0)
    h = (jnp.dot(a0, w_ref[0], preferred_element_type=jnp.float32)
         + jnp.dot(a1, w_ref[1], preferred_element_type=jnp.float32))
    out_ref[0] = h * dinv


def _mm2_call(agg1, dinv, b1r, W2r):
    return pl.pallas_call(
        _mm2_body,
        grid=(N // _BM, 2),
        in_specs=[
            pl.BlockSpec((2, _BM, HH), lambda i, j: (0, i, 0)),
            pl.BlockSpec((_BM, 1), lambda i, j: (i, 0)),
            pl.BlockSpec((2, HH), lambda i, j: (0, 0)),
            pl.BlockSpec((2, HH, HH), lambda i, j: (0, 0, j)),
        ],
        out_specs=pl.BlockSpec((1, _BM, HH), lambda i, j: (j, i, 0)),
        out_shape=jax.ShapeDtypeStruct((2, N, HH), jnp.float32),
    )(agg1, dinv, b1r, W2r)


def _out_body(agg_ref, dinv_ref, b_ref, out_ref):
    j = pl.program_id(1)
    out_ref[...] = jnp.maximum(
        agg_ref[0] * dinv_ref[...] + b_ref[pl.ds(j, 1)], 0.0)


def _out_call(agg2, dinv, b2r):
    return pl.pallas_call(
        _out_body,
        grid=(N // _BM, 2),
        in_specs=[
            pl.BlockSpec((1, _BM, HH), lambda i, j: (j, i, 0)),
            pl.BlockSpec((_BM, 1), lambda i, j: (i, 0)),
            pl.BlockSpec((2, HH), lambda i, j: (0, 0)),
        ],
        out_specs=pl.BlockSpec((_BM, HH), lambda i, j: (i, j)),
        out_shape=jax.ShapeDtypeStruct((N, 256), jnp.float32),
    )(agg2, dinv, b2r)


def _pad_rows(h_split):
    return jnp.pad(h_split, ((0, 0), (0, NPAD - N), (0, 0)))


def kernel(x, edge_index, W1, b1, W2, b2):
    ei = edge_index.astype(jnp.int32)
    src = ei[0].reshape(NTILES, AGG_CHUNKS, CHUNK)
    dst = ei[1].reshape(NTILES, AGG_CHUNKS, CHUNK)
    dst_deg = ei[1].reshape(2, NTILES, DEG_CHUNKS, CHUNK)
    zeros = jnp.zeros((RPT,), jnp.float32)
    ones = jnp.ones((CHUNK,), jnp.float32)

    degp = _deg_kernel(dst_deg, zeros, ones)
    dinv = lax.rsqrt(degp[0, :N] + degp[1, :N] + 1.0).reshape(N, 1)
    h1p = _mm1_call(dinv, x, W1)
    agg1 = _agg_kernel(_pad_rows(h1p), src, dst)[:, :N]
    h2p = _mm2_call(agg1, dinv, b1.reshape(2, HH), W2.reshape(2, HH, 256))
    agg2 = _agg_kernel(_pad_rows(h2p), src, dst)[:, :N]
    return _out_call(agg2, dinv, b2.reshape(2, HH))
